# Initial kernel scaffold; baseline (speedup 1.0000x reference)
import jax
import jax.numpy as jnp
from jax.experimental import pallas as pl


def kernel(x, w):
    raise NotImplementedError("write your pallas kernel here")


# speedup vs baseline: 19.8290x; 19.8290x over previous
import functools

import jax
import jax.numpy as jnp
from jax import lax
from jax.experimental import pallas as pl
from jax.experimental.pallas import tpu as pltpu

_C = 384
_K = _C // 2
_BISECT_ITERS = 26


def _ptap_body(w_ref, x_ref, o_ref):
    xb = x_ref[0]
    y = jnp.mean(xb, axis=1, keepdims=True)
    z = jnp.zeros((1, 1), dtype=y.dtype)
    y_prev = jnp.concatenate([z, y[:-1]], axis=0)
    y_next = jnp.concatenate([y[1:], z], axis=0)
    conv = y_prev * w_ref[0] + y * w_ref[1] + y_next * w_ref[2]
    att = jax.nn.sigmoid(conv)
    fw = xb * att

    lo = jnp.min(fw, axis=0, keepdims=True)
    hi = jnp.max(fw, axis=0, keepdims=True)
    kf = jnp.float32(_K)

    def step(_, carry):
        lo, hi = carry
        mid = 0.5 * (lo + hi)
        cnt = jnp.sum((fw >= mid).astype(jnp.float32), axis=0, keepdims=True)
        pred = cnt >= kf
        return jnp.where(pred, mid, lo), jnp.where(pred, hi, mid)

    lo, hi = lax.fori_loop(0, _BISECT_ITERS, step, (lo, hi))
    t = lo
    s = jnp.sum(jnp.maximum(fw - t, 0.0), axis=0, keepdims=True)
    o_ref[0] = (s + kf * t) * (1.0 / kf)


def kernel(x, w):
    B, C, H, W = x.shape
    P = H * W
    xr = x.reshape(B, C, P)
    out = pl.pallas_call(
        _ptap_body,
        grid=(B,),
        in_specs=[
            pl.BlockSpec(memory_space=pltpu.SMEM),
            pl.BlockSpec((1, C, P), lambda b: (b, 0, 0)),
        ],
        out_specs=pl.BlockSpec((1, 1, P), lambda b: (b, 0, 0)),
        out_shape=jax.ShapeDtypeStruct((B, 1, P), jnp.float32),
    )(w, xr)
    return out.reshape(B, H, W)


# speedup vs baseline: 28.7605x; 1.4504x over previous
import functools

import jax
import jax.numpy as jnp
from jax import lax
from jax.experimental import pallas as pl
from jax.experimental.pallas import tpu as pltpu

_C = 384
_K = _C // 2
_BISECT_ITERS = 14


def _ptap_body(w_ref, x_ref, o_ref):
    xb = x_ref[0]
    y = jnp.mean(xb, axis=1, keepdims=True)
    z = jnp.zeros((1, 1), dtype=y.dtype)
    y_prev = jnp.concatenate([z, y[:-1]], axis=0)
    y_next = jnp.concatenate([y[1:], z], axis=0)
    conv = y_prev * w_ref[0] + y * w_ref[1] + y_next * w_ref[2]
    att = jax.nn.sigmoid(conv)
    fw = xb * att

    lo = jnp.min(fw, axis=0, keepdims=True)
    hi = jnp.max(fw, axis=0, keepdims=True)
    kf = jnp.float32(_K)

    def step(_, carry):
        lo, hi = carry
        mid = 0.5 * (lo + hi)
        cnt = jnp.sum((fw >= mid).astype(jnp.float32), axis=0, keepdims=True)
        pred = cnt >= kf
        return jnp.where(pred, mid, lo), jnp.where(pred, hi, mid)

    lo, hi = lax.fori_loop(0, _BISECT_ITERS, step, (lo, hi))
    t = lo
    s = jnp.sum(jnp.maximum(fw - t, 0.0), axis=0, keepdims=True)
    o_ref[0] = (s + kf * t) * (1.0 / kf)


def kernel(x, w):
    B, C, H, W = x.shape
    P = H * W
    xr = x.reshape(B, C, P)
    out = pl.pallas_call(
        _ptap_body,
        grid=(B,),
        in_specs=[
            pl.BlockSpec(memory_space=pltpu.SMEM),
            pl.BlockSpec((1, C, P), lambda b: (b, 0, 0)),
        ],
        out_specs=pl.BlockSpec((1, 1, P), lambda b: (b, 0, 0)),
        out_shape=jax.ShapeDtypeStruct((B, 1, P), jnp.float32),
    )(w, xr)
    return out.reshape(B, H, W)
